# baseline (speedup 1.0000x reference)
import functools

import jax
import jax.numpy as jnp
from jax import lax
from jax.experimental import pallas as pl
from jax.experimental.pallas import tpu as pltpu
from jax.experimental.pallas import tpu_sc as plsc

_D = 64
_EPS = 1e-12


def _sc_gather(token_table, input_ids):
    b, s = input_ids.shape
    info = plsc.get_sparse_core_info()
    nw = info.num_cores * info.num_subcores
    bpw = b // nw
    bc = 4
    n_chunks = bpw // bc
    s_splits = []
    off = 0
    while off < s:
        w = min(128, s - off)
        s_splits.append((off, w))
        off += w
    mesh = plsc.VectorSubcoreMesh(core_axis_name="c", subcore_axis_name="s")

    @functools.partial(
        pl.kernel,
        mesh=mesh,
        compiler_params=pltpu.CompilerParams(use_tc_tiling_on_sc=False),
        out_type=jax.ShapeDtypeStruct((b, s, _D), jnp.float32),
        scratch_types=[
            pltpu.VMEM((bc, s), jnp.int32),
            pltpu.VMEM((bc, s, _D), jnp.float32),
            pltpu.SemaphoreType.DMA,
        ],
    )
    def k(table_hbm, idx_hbm, out_hbm, idx_v, rows_v, sem):
        cid = lax.axis_index("c")
        sid = lax.axis_index("s")
        wid = sid * info.num_cores + cid

        def chunk(g, carry):
            b0 = wid * bpw + g * bc
            pltpu.sync_copy(idx_hbm.at[pl.ds(b0, bc)], idx_v)
            copies = []
            for i in range(bc):
                for off, w in s_splits:
                    copies.append(
                        pltpu.async_copy(
                            table_hbm.at[idx_v.at[i, pl.ds(off, w)]],
                            rows_v.at[i, pl.ds(off, w)],
                            sem,
                        )
                    )
            for c in copies:
                c.wait()
            pltpu.sync_copy(rows_v, out_hbm.at[pl.ds(b0, bc)])
            return carry

        lax.fori_loop(0, n_chunks, chunk, 0)

    return k(token_table, input_ids)


def _tc_layernorm(gathered3d, pos3d, gamma3d, beta3d):
    b, s, d = gathered3d.shape
    bb = 32

    def body(x_ref, pos_ref, gamma_ref, beta_ref, o_ref):
        x = x_ref[...] + pos_ref[...]
        mean = jnp.mean(x, axis=-1, keepdims=True)
        xc = x - mean
        var = jnp.mean(xc * xc, axis=-1, keepdims=True)
        o_ref[...] = (
            xc * lax.rsqrt(var + _EPS) * gamma_ref[...] + beta_ref[...]
        )

    return pl.pallas_call(
        body,
        grid=(b // bb,),
        in_specs=[
            pl.BlockSpec((bb, s, d), lambda i: (i, 0, 0)),
            pl.BlockSpec((1, s, d), lambda i: (0, 0, 0)),
            pl.BlockSpec((1, 1, d), lambda i: (0, 0, 0)),
            pl.BlockSpec((1, 1, d), lambda i: (0, 0, 0)),
        ],
        out_specs=pl.BlockSpec((bb, s, d), lambda i: (i, 0, 0)),
        out_shape=jax.ShapeDtypeStruct((b, s, d), jnp.float32),
    )(gathered3d, pos3d, gamma3d, beta3d)


def kernel(input_ids, token_table, pos_table, gamma, beta):
    b, s = input_ids.shape
    gathered = _sc_gather(token_table, input_ids)
    return _tc_layernorm(
        gathered,
        pos_table.reshape(1, s, _D),
        gamma.reshape(1, 1, _D),
        beta.reshape(1, 1, _D),
    )


# speedup vs baseline: 1.0048x; 1.0048x over previous
import functools

import jax
import jax.numpy as jnp
from jax import lax
from jax.experimental import pallas as pl
from jax.experimental.pallas import tpu as pltpu
from jax.experimental.pallas import tpu_sc as plsc

_D = 64
_EPS = 1e-12


_G = 128
_F = 1280


def _sc_gather(token_table, idx2d, n_rows):
    info = plsc.get_sparse_core_info()
    nw = info.num_cores * info.num_subcores
    per_w = n_rows // nw
    n_chunks = per_w // _F
    mesh = plsc.VectorSubcoreMesh(core_axis_name="c", subcore_axis_name="s")

    @functools.partial(
        pl.kernel,
        mesh=mesh,
        compiler_params=pltpu.CompilerParams(use_tc_tiling_on_sc=False),
        out_type=jax.ShapeDtypeStruct((n_rows, _D), jnp.float32),
        scratch_types=[
            pltpu.VMEM((_F // _G, _G), jnp.int32),
            pltpu.VMEM((_F, _D), jnp.float32),
            pltpu.SemaphoreType.DMA,
        ],
    )
    def k(table_hbm, idx_hbm, out_hbm, idx_v, rows_v, sem):
        cid = lax.axis_index("c")
        sid = lax.axis_index("s")
        wid = sid * info.num_cores + cid

        def chunk(g, carry):
            base = wid * per_w + g * _F
            pltpu.sync_copy(idx_hbm.at[pl.ds(base // _G, _F // _G)], idx_v)
            copies = []
            for j in range(_F // _G):
                copies.append(
                    pltpu.async_copy(
                        table_hbm.at[idx_v.at[j]],
                        rows_v.at[pl.ds(j * _G, _G)],
                        sem,
                    )
                )
            for c in copies:
                c.wait()
            pltpu.sync_copy(rows_v, out_hbm.at[pl.ds(base, _F)])
            return carry

        lax.fori_loop(0, n_chunks, chunk, 0)

    return k(token_table, idx2d)


def _tc_layernorm(gathered3d, pos3d, gamma3d, beta3d):
    b, s, d = gathered3d.shape
    bb = 32

    def body(x_ref, pos_ref, gamma_ref, beta_ref, o_ref):
        x = x_ref[...] + pos_ref[...]
        mean = jnp.mean(x, axis=-1, keepdims=True)
        xc = x - mean
        var = jnp.mean(xc * xc, axis=-1, keepdims=True)
        o_ref[...] = (
            xc * lax.rsqrt(var + _EPS) * gamma_ref[...] + beta_ref[...]
        )

    return pl.pallas_call(
        body,
        grid=(b // bb,),
        in_specs=[
            pl.BlockSpec((bb, s, d), lambda i: (i, 0, 0)),
            pl.BlockSpec((1, s, d), lambda i: (0, 0, 0)),
            pl.BlockSpec((1, 1, d), lambda i: (0, 0, 0)),
            pl.BlockSpec((1, 1, d), lambda i: (0, 0, 0)),
        ],
        out_specs=pl.BlockSpec((bb, s, d), lambda i: (i, 0, 0)),
        out_shape=jax.ShapeDtypeStruct((b, s, d), jnp.float32),
    )(gathered3d, pos3d, gamma3d, beta3d)


def kernel(input_ids, token_table, pos_table, gamma, beta):
    b, s = input_ids.shape
    n_rows = b * s
    idx2d = input_ids.reshape(n_rows // _G, _G)
    gathered = _sc_gather(token_table, idx2d, n_rows)
    return _tc_layernorm(
        gathered.reshape(b, s, _D),
        pos_table.reshape(1, s, _D),
        gamma.reshape(1, 1, _D),
        beta.reshape(1, 1, _D),
    )
